# Initial kernel scaffold; baseline (speedup 1.0000x reference)
import jax
import jax.numpy as jnp
from jax.experimental import pallas as pl


def kernel(x, h, J, edge_idx_i, edge_idx_j):
    raise NotImplementedError("write your pallas kernel here")


# speedup vs baseline: 1.8843x; 1.8843x over previous
import functools

import jax
import jax.numpy as jnp
from jax import lax
from jax.experimental import pallas as pl
from jax.experimental.pallas import tpu as pltpu
from jax.experimental.pallas import tpu_sc as plsc

B = 1024
N = 10000
E = 160000

NC = 2
NS = 16
NW = NC * NS
ROWS_PER_W = B // NW
SLAB = 8
N_SLABS = ROWS_PER_W // SLAB
CHUNK = 8000
N_CHUNKS = E // CHUNK
GROUPS = CHUNK // 16
H_GROUPS = N // 16

_mesh = plsc.VectorSubcoreMesh(core_axis_name="c", subcore_axis_name="s")


@functools.partial(
    pl.kernel,
    mesh=_mesh,
    compiler_params=pltpu.CompilerParams(needs_layout_passes=False),
    out_type=jax.ShapeDtypeStruct((B,), jnp.float32),
    scratch_types=[
        pltpu.VMEM((SLAB * N,), jnp.float32),
        pltpu.VMEM((N,), jnp.float32),
        pltpu.VMEM((CHUNK,), jnp.int32),
        pltpu.VMEM((CHUNK,), jnp.int32),
        pltpu.VMEM((CHUNK,), jnp.float32),
        pltpu.VMEM((ROWS_PER_W,), jnp.float32),
    ],
)
def _rbm_sc(x_hbm, h_hbm, j_hbm, ei_hbm, ej_hbm, out_hbm,
            xslab, h_v, ei_v, ej_v, jv_v, out_v):
    wid = lax.axis_index("s") * NC + lax.axis_index("c")

    pltpu.sync_copy(h_hbm, h_v)

    lane = lax.iota(jnp.int32, 16)

    sums = []
    for s in range(N_SLABS):
        row0 = (wid * N_SLABS + s) * SLAB
        pltpu.sync_copy(x_hbm.at[pl.ds(row0 * N, SLAB * N)], xslab)

        def h_body(k, accs):
            base = k * 16
            hv = h_v[pl.ds(base, 16)]
            return tuple(
                accs[r] + xslab[pl.ds(r * N + base, 16)] * hv
                for r in range(SLAB)
            )

        accs = tuple(jnp.zeros((16,), jnp.float32) for _ in range(SLAB))
        accs = lax.fori_loop(0, H_GROUPS, h_body, accs)

        def chunk_body(c, accs):
            off = c * CHUNK
            pltpu.sync_copy(ei_hbm.at[pl.ds(off, CHUNK)], ei_v)
            pltpu.sync_copy(ej_hbm.at[pl.ds(off, CHUNK)], ej_v)
            pltpu.sync_copy(j_hbm.at[pl.ds(off, CHUNK)], jv_v)

            def group_body(g, accs):
                base = g * 16
                ii = ei_v[pl.ds(base, 16)]
                jj = ej_v[pl.ds(base, 16)]
                Jv = jv_v[pl.ds(base, 16)]
                new = []
                for r in range(SLAB):
                    vi = plsc.load_gather(xslab, [ii + (r * N)])
                    vj = plsc.load_gather(xslab, [jj + (r * N)])
                    new.append(accs[r] + vi * vj * Jv)
                return tuple(new)

            return lax.fori_loop(0, GROUPS, group_body, accs)

        accs = lax.fori_loop(0, N_CHUNKS, chunk_body, accs)
        sums.extend(lax.reduce_sum_p.bind(a, axes=(0,)) for a in accs)

    for half in range(ROWS_PER_W // 16):
        vec = jnp.zeros((16,), jnp.float32)
        for k in range(16):
            vec = jnp.where(lane == k, sums[half * 16 + k], vec)
        out_v[pl.ds(half * 16, 16)] = vec
    pltpu.sync_copy(out_v, out_hbm.at[pl.ds(wid * ROWS_PER_W, ROWS_PER_W)])


def kernel(x, h, J, edge_idx_i, edge_idx_j):
    return _rbm_sc(x.reshape(-1), h, J, edge_idx_i, edge_idx_j)


# speedup vs baseline: 2.3740x; 1.2599x over previous
import functools

import jax
import jax.numpy as jnp
from jax import lax
from jax.experimental import pallas as pl
from jax.experimental.pallas import tpu as pltpu
from jax.experimental.pallas import tpu_sc as plsc

B = 1024
N = 10000
E = 160000

NC = 2
NS = 16
NW = NC * NS
ROWS_PER_W = B // NW
SLAB = 8
N_SLABS = ROWS_PER_W // SLAB
CHUNK = 4000
N_PAIRS = E // (2 * CHUNK)
GROUPS = CHUNK // 16
H_GROUPS = N // 16

_mesh = plsc.VectorSubcoreMesh(core_axis_name="c", subcore_axis_name="s")


@functools.partial(
    pl.kernel,
    mesh=_mesh,
    compiler_params=pltpu.CompilerParams(needs_layout_passes=False),
    out_type=jax.ShapeDtypeStruct((B,), jnp.float32),
    scratch_types=[
        pltpu.VMEM((SLAB * N,), jnp.float32),
        pltpu.VMEM((N,), jnp.float32),
        pltpu.VMEM((CHUNK,), jnp.int32),
        pltpu.VMEM((CHUNK,), jnp.int32),
        pltpu.VMEM((CHUNK,), jnp.float32),
        pltpu.VMEM((CHUNK,), jnp.int32),
        pltpu.VMEM((CHUNK,), jnp.int32),
        pltpu.VMEM((CHUNK,), jnp.float32),
        pltpu.VMEM((ROWS_PER_W,), jnp.float32),
        pltpu.SemaphoreType.DMA,
        pltpu.SemaphoreType.DMA,
    ],
)
def _rbm_sc(x_hbm, h_hbm, j_hbm, ei_hbm, ej_hbm, out_hbm,
            xslab, h_v, ei_a, ej_a, jv_a, ei_b, ej_b, jv_b, out_v,
            sem_a, sem_b):
    wid = lax.axis_index("s") * NC + lax.axis_index("c")

    pltpu.sync_copy(h_hbm, h_v)

    lane = lax.iota(jnp.int32, 16)

    def start_chunk(c, bufs, sem):
        off = c * CHUNK
        pltpu.async_copy(ei_hbm.at[pl.ds(off, CHUNK)], bufs[0], sem)
        pltpu.async_copy(ej_hbm.at[pl.ds(off, CHUNK)], bufs[1], sem)
        pltpu.async_copy(j_hbm.at[pl.ds(off, CHUNK)], bufs[2], sem)

    def wait_chunk(bufs, sem):
        pltpu.make_async_copy(ei_hbm.at[pl.ds(0, CHUNK)], bufs[0], sem).wait()
        pltpu.make_async_copy(ej_hbm.at[pl.ds(0, CHUNK)], bufs[1], sem).wait()
        pltpu.make_async_copy(j_hbm.at[pl.ds(0, CHUNK)], bufs[2], sem).wait()

    def edge_accum(bufs, accs):
        ei_v, ej_v, jv_v = bufs

        def group_body(g, accs):
            base = g * 16
            ii = ei_v[pl.ds(base, 16)]
            jj = ej_v[pl.ds(base, 16)]
            Jv = jv_v[pl.ds(base, 16)]
            new = []
            for r in range(SLAB):
                vi = plsc.load_gather(xslab, [ii + (r * N)])
                vj = plsc.load_gather(xslab, [jj + (r * N)])
                new.append(accs[r] + vi * vj * Jv)
            return tuple(new)

        return plsc.parallel_loop(0, GROUPS, unroll=2, carry=accs)(group_body)

    bufs_a = (ei_a, ej_a, jv_a)
    bufs_b = (ei_b, ej_b, jv_b)

    sums = []
    for s in range(N_SLABS):
        row0 = (wid * N_SLABS + s) * SLAB
        pltpu.async_copy(x_hbm.at[pl.ds(row0 * N, SLAB * N)], xslab, sem_a)
        start_chunk(0, bufs_a, sem_a)
        pltpu.make_async_copy(
            x_hbm.at[pl.ds(0, SLAB * N)], xslab, sem_a).wait()

        def h_body(k, accs):
            base = k * 16
            hv = h_v[pl.ds(base, 16)]
            return tuple(
                accs[r] + xslab[pl.ds(r * N + base, 16)] * hv
                for r in range(SLAB)
            )

        accs = tuple(jnp.zeros((16,), jnp.float32) for _ in range(SLAB))
        accs = lax.fori_loop(0, H_GROUPS, h_body, accs)

        def pair_body(c, accs):
            start_chunk(2 * c + 1, bufs_b, sem_b)
            wait_chunk(bufs_a, sem_a)
            accs = edge_accum(bufs_a, accs)

            @pl.when(c < N_PAIRS - 1)
            def _():
                start_chunk(2 * c + 2, bufs_a, sem_a)

            wait_chunk(bufs_b, sem_b)
            return edge_accum(bufs_b, accs)

        accs = lax.fori_loop(0, N_PAIRS, pair_body, accs)
        sums.extend(lax.reduce_sum_p.bind(a, axes=(0,)) for a in accs)

    for half in range(ROWS_PER_W // 16):
        vec = jnp.zeros((16,), jnp.float32)
        for k in range(16):
            vec = jnp.where(lane == k, sums[half * 16 + k], vec)
        out_v[pl.ds(half * 16, 16)] = vec
    pltpu.sync_copy(out_v, out_hbm.at[pl.ds(wid * ROWS_PER_W, ROWS_PER_W)])


def kernel(x, h, J, edge_idx_i, edge_idx_j):
    return _rbm_sc(x.reshape(-1), h, J, edge_idx_i, edge_idx_j)
